# baseline (speedup 1.0000x reference)
import jax
import jax.numpy as jnp
from jax.experimental import pallas as pl

B = 32
N_J = 50
N_M = 20
N = N_J * N_M
D = 64
H = 64
G = 2

_NEG_INF = float("-inf")

_WD01, _WD02, _WD11, _WD12 = 0, 128, 256, 384
_A1C, _A1H, _A1M, _AW2, _CW1 = 0, 64, 128, 192, 256
_B01, _B02, _B11, _B12 = 320, 321, 322, 323
_AB1, _AB2, _CB1, _AW3, _CW2, _MISC = 324, 325, 326, 327, 328, 329


def _body(adj_ref, x_ref, gm_ref, idx_ref, maskf_ref, dur_ref, mm_ref,
          wd_ref, ws_ref,
          ent_ref, v_ref, loga_ref, anode_ref, afeat_ref, mma_ref,
          hpool_ref):
    f32 = jnp.float32
    bf16 = jnp.bfloat16

    adj_s = adj_ref[...].reshape(G * N, N).astype(bf16)
    xpair = jnp.concatenate([x_ref[0], x_ref[1]],
                            axis=1).astype(bf16)
    f0 = jnp.dot(adj_s, xpair, preferred_element_type=f32)
    t0 = jnp.maximum(jnp.dot(f0.astype(bf16), wd_ref[_WD01:_WD01 + 128, :],
                             preferred_element_type=f32)
                     + ws_ref[_B01:_B01 + 1, :], 0.0)
    h1 = jnp.maximum(jnp.dot(t0.astype(bf16), wd_ref[_WD02:_WD02 + 128, :],
                             preferred_element_type=f32)
                     + ws_ref[_B02:_B02 + 1, :], 0.0)

    lane = jax.lax.broadcasted_iota(jnp.int32, (N, 2 * H), 1)
    h1pair = jnp.where(lane < H, h1[0:N, :], h1[N:2 * N, :]).astype(bf16)
    f1 = jnp.dot(adj_s, h1pair, preferred_element_type=f32)
    t1 = jnp.maximum(jnp.dot(f1.astype(bf16), wd_ref[_WD11:_WD11 + 128, :],
                             preferred_element_type=f32)
                     + ws_ref[_B11:_B11 + 1, :], 0.0)
    h2 = jnp.maximum(jnp.dot(t1.astype(bf16), wd_ref[_WD12:_WD12 + 128, :],
                             preferred_element_type=f32)
                     + ws_ref[_B12:_B12 + 1, :], 0.0)

    for g in range(G):
        h2g = h2[g * N:(g + 1) * N, g * H:(g + 1) * H]
        gp = gm_ref[g][:, 0:N]
        mch = gm_ref[g][:, N:N + H]
        h_pooled = jnp.dot(gp, h2g, preferred_element_type=f32)

        cand = idx_ref[g][0:N_J, :]
        iota_n = jax.lax.broadcasted_iota(jnp.int32, (N_J, N), 1)
        onehot = (iota_n == cand).astype(f32)
        cand_feat = jnp.dot(onehot, h2g, preferred_element_type=f32)

        a1 = jnp.tanh(
            jnp.dot(cand_feat, ws_ref[_A1C:_A1C + H, 0:H],
                    preferred_element_type=f32)
            + jnp.dot(h_pooled, ws_ref[_A1H:_A1H + H, 0:H],
                      preferred_element_type=f32)
            + jnp.dot(mch, ws_ref[_A1M:_A1M + H, 0:H],
                      preferred_element_type=f32)
            + ws_ref[_AB1:_AB1 + 1, 0:H])
        a2 = jnp.tanh(jnp.dot(a1, ws_ref[_AW2:_AW2 + H, 0:H],
                              preferred_element_type=f32)
                      + ws_ref[_AB2:_AB2 + 1, 0:H])
        ab3 = ws_ref[_MISC:_MISC + 1, 0:1]
        scores = (jnp.sum(a2 * ws_ref[_AW3:_AW3 + 1, 0:H], axis=1,
                          keepdims=True) + ab3) * 10.0
        scores = jnp.where(maskf_ref[g] > 0.5, _NEG_INF, scores)

        m = jnp.max(scores, axis=0, keepdims=True)
        e = jnp.exp(scores - m)
        z = jnp.sum(e, axis=0, keepdims=True)
        log_pi = scores - m - jnp.log(z)
        pi = jnp.exp(log_pi)
        ent_ref[g] = -jnp.sum(pi * log_pi, axis=0, keepdims=True)

        aidx = idx_ref[g][N_J:N_J + 1, :]
        iota_j = jax.lax.broadcasted_iota(jnp.int32, (N_J, 1), 0)
        oh_a = (iota_j == aidx).astype(f32)
        loga_ref[g] = jnp.sum(log_pi * oh_a, axis=0, keepdims=True)

        oa = idx_ref[g][N_J + 1:N_J + 2, :]
        iota_row = jax.lax.broadcasted_iota(jnp.int32, (1, N), 1)
        oh_o = (iota_row == oa).astype(f32)
        afeat_ref[g] = jnp.dot(oh_o, h2g, preferred_element_type=f32)
        anode_ref[g] = jnp.dot(oh_o, dur_ref[g], preferred_element_type=f32)
        mmf = mm_ref[g].astype(f32)
        mma_ref[g] = jnp.dot(oh_o, mmf, preferred_element_type=f32) > 0.5

        c1 = jnp.tanh(jnp.dot(h_pooled, ws_ref[_CW1:_CW1 + H, 0:H],
                              preferred_element_type=f32)
                      + ws_ref[_CB1:_CB1 + 1, 0:H])
        cb2 = ws_ref[_MISC:_MISC + 1, 1:2]
        v_ref[g] = (jnp.sum(c1 * ws_ref[_CW2:_CW2 + 1, 0:H], axis=1,
                            keepdims=True) + cb2)
        hpool_ref[g] = h_pooled


def _b3(shape):
    return pl.BlockSpec((G,) + shape, lambda b: (b, 0, 0))


def _w(shape):
    return pl.BlockSpec(shape, lambda b: (0,) * len(shape))


def kernel(x, graph_pool, padded_nei, adj, candidate, mask, mask_mch, dur,
           a_index, old_action, mch_pool,
           gW01, gb01, gW02, gb02, gW11, gb11, gW12, gb12,
           aW1, ab1, aW2, ab2, aW3, ab3, cW1, cb1, cW2, cb2):
    f32 = jnp.float32
    bf16 = jnp.bfloat16
    i32 = jnp.int32

    zw = jnp.zeros((D, H), f32)

    def bd(w):
        return jnp.concatenate(
            [jnp.concatenate([w, zw], 1), jnp.concatenate([zw, w], 1)], 0)

    wd = jnp.concatenate(
        [bd(gW01), bd(gW02), bd(gW11), bd(gW12)], 0).astype(bf16)

    def pad128(w):
        return jnp.concatenate([w, jnp.zeros_like(w)], 1)

    def row128(vec64):
        return jnp.concatenate([vec64, jnp.zeros((H,), f32)])[None, :]

    misc = jnp.concatenate([ab3, cb2, jnp.zeros((126,), f32)])[None, :]
    ws = jnp.concatenate(
        [pad128(aW1), pad128(aW2), pad128(cW1),
         jnp.concatenate([gb01, gb01])[None, :],
         jnp.concatenate([gb02, gb02])[None, :],
         jnp.concatenate([gb11, gb11])[None, :],
         jnp.concatenate([gb12, gb12])[None, :],
         row128(ab1), row128(ab2), row128(cb1),
         row128(aW3[:, 0]), row128(cW2[:, 0]), misc], 0)

    gm = jnp.concatenate([graph_pool, mch_pool], axis=1)[:, None, :]
    idx_all = jnp.concatenate(
        [candidate.astype(i32), a_index.astype(i32)[:, None],
         old_action.astype(i32)[:, None]], axis=1)[:, :, None]
    maskf3 = mask.astype(f32)[:, :, None]

    out_shapes = (
        jax.ShapeDtypeStruct((B, 1, 1), f32),
        jax.ShapeDtypeStruct((B, 1, 1), f32),
        jax.ShapeDtypeStruct((B, 1, 1), f32),
        jax.ShapeDtypeStruct((B, 1, N_M), f32),
        jax.ShapeDtypeStruct((B, 1, H), f32),
        jax.ShapeDtypeStruct((B, 1, N_M), jnp.bool_),
        jax.ShapeDtypeStruct((B, 1, H), f32),
    )
    in_specs = [
        _b3((N, N)),
        _b3((N, D)),
        _b3((1, N + H)),
        _b3((N_J + 2, 1)),
        _b3((N_J, 1)),
        _b3((N, N_M)),
        _b3((N, N_M)),
        _w((512, 128)),
        _w((330, 128)),
    ]
    out_specs = (
        _b3((1, 1)), _b3((1, 1)), _b3((1, 1)), _b3((1, N_M)),
        _b3((1, H)), _b3((1, N_M)), _b3((1, H)),
    )
    ent, v, loga, anode, afeat, mma, hpool = pl.pallas_call(
        _body,
        grid=(B // G,),
        in_specs=in_specs,
        out_specs=out_specs,
        out_shape=out_shapes,
    )(adj, x, gm, idx_all, maskf3, dur, mask_mch, wd, ws)

    return (ent.reshape(B), v.reshape(B, 1), loga.reshape(B),
            anode.reshape(B, N_M), afeat.reshape(B, H),
            mma, hpool.reshape(B, H))


# speedup vs baseline: 1.1289x; 1.1289x over previous
import functools

import jax
import jax.numpy as jnp
from jax import lax
from jax.experimental import pallas as pl
from jax.experimental.pallas import tpu as pltpu
from jax.experimental.pallas import tpu_sc as plsc

B = 32
N_J = 50
N_M = 20
N = N_J * N_M
D = 64
H = 64
G = 2

_NEG_INF = float("-inf")

_WD01, _WD02, _WD11, _WD12 = 0, 128, 256, 384
_A1C, _A1H, _A1M, _AW2, _CW1 = 0, 64, 128, 192, 256
_B01, _B02, _B11, _B12 = 320, 321, 322, 323
_AB1, _AB2, _CB1, _AW3, _CW2, _MISC = 324, 325, 326, 327, 328, 329

_RPW = 8
_NW = B // _RPW
_PAD_W = 128


def _sc_gather(table, idx):
    mesh = plsc.VectorSubcoreMesh(core_axis_name="c", subcore_axis_name="s")

    @functools.partial(
        pl.kernel, mesh=mesh,
        out_type=jax.ShapeDtypeStruct((B, _PAD_W), jnp.float32),
        scratch_types=[
            pltpu.VMEM((_RPW,), jnp.int32),
            pltpu.VMEM((_RPW, _PAD_W), jnp.float32),
            pltpu.SemaphoreType.DMA,
        ],
    )
    def k(table_hbm, idx_hbm, out_hbm, idx_v, rows_v, sem):
        wid = lax.axis_index("s") * 2 + lax.axis_index("c")

        @pl.when(wid < _NW)
        def _():
            base = wid * _RPW
            pltpu.sync_copy(idx_hbm.at[pl.ds(base, _RPW)], idx_v)
            pltpu.async_copy(table_hbm.at[idx_v], rows_v, sem).wait()
            pltpu.sync_copy(rows_v, out_hbm.at[pl.ds(base, _RPW)])

    return k(table, idx)


def _sc_action_rows(dur, mask_mch, old_action):
    f32 = jnp.float32
    i32 = jnp.int32
    dur2 = dur.reshape(B * N, N_M)
    mm_w = jax.lax.bitcast_convert_type(
        mask_mch.view(jnp.int8).reshape(B * N, N_M // 4, 4), i32)
    table = jnp.concatenate(
        [dur2, jax.lax.bitcast_convert_type(mm_w, f32),
         jnp.zeros((B * N, _PAD_W - N_M - N_M // 4), f32)], axis=1)
    ridx = old_action.astype(i32) + jnp.arange(B, dtype=i32) * N
    rows = _sc_gather(table, ridx)
    action_node = rows[:, :N_M]
    mm_back = jax.lax.bitcast_convert_type(
        jax.lax.bitcast_convert_type(rows[:, N_M:N_M + N_M // 4], i32),
        jnp.int8).reshape(B, 1, N_M)
    return action_node, mm_back != 0


def _body(adj_ref, x_ref, gm_ref, idx_ref, maskf_ref,
          wd_ref, ws_ref,
          ent_ref, v_ref, loga_ref, afeat_ref, hpool_ref):
    f32 = jnp.float32
    bf16 = jnp.bfloat16

    adjs = [adj_ref[g].astype(bf16) for g in range(G)]
    p0s = [jnp.dot(adjs[g], x_ref[g].astype(bf16),
                   preferred_element_type=f32) for g in range(G)]
    p0 = jnp.concatenate(p0s, axis=1)

    t0 = jnp.maximum(jnp.dot(p0.astype(bf16), wd_ref[_WD01:_WD01 + 128, :],
                             preferred_element_type=f32)
                     + ws_ref[_B01:_B01 + 1, :], 0.0)
    h1 = jnp.maximum(jnp.dot(t0.astype(bf16), wd_ref[_WD02:_WD02 + 128, :],
                             preferred_element_type=f32)
                     + ws_ref[_B02:_B02 + 1, :], 0.0)

    p1s = [jnp.dot(adjs[g], h1[:, g * H:(g + 1) * H].astype(bf16),
                   preferred_element_type=f32) for g in range(G)]
    p1 = jnp.concatenate(p1s, axis=1)
    t1 = jnp.maximum(jnp.dot(p1.astype(bf16), wd_ref[_WD11:_WD11 + 128, :],
                             preferred_element_type=f32)
                     + ws_ref[_B11:_B11 + 1, :], 0.0)
    h2 = jnp.maximum(jnp.dot(t1.astype(bf16), wd_ref[_WD12:_WD12 + 128, :],
                             preferred_element_type=f32)
                     + ws_ref[_B12:_B12 + 1, :], 0.0)

    for g in range(G):
        h2g = h2[:, g * H:(g + 1) * H]
        gp = gm_ref[g][:, 0:N]
        mch = gm_ref[g][:, N:N + H]
        h_pooled = jnp.dot(gp, h2g, preferred_element_type=f32)

        cand = idx_ref[g][0:N_J, :]
        iota_n = jax.lax.broadcasted_iota(jnp.int32, (N_J, N), 1)
        onehot = (iota_n == cand).astype(f32)
        cand_feat = jnp.dot(onehot, h2g, preferred_element_type=f32)

        a1 = jnp.tanh(
            jnp.dot(cand_feat, ws_ref[_A1C:_A1C + H, 0:H],
                    preferred_element_type=f32)
            + jnp.dot(h_pooled, ws_ref[_A1H:_A1H + H, 0:H],
                      preferred_element_type=f32)
            + jnp.dot(mch, ws_ref[_A1M:_A1M + H, 0:H],
                      preferred_element_type=f32)
            + ws_ref[_AB1:_AB1 + 1, 0:H])
        a2 = jnp.tanh(jnp.dot(a1, ws_ref[_AW2:_AW2 + H, 0:H],
                              preferred_element_type=f32)
                      + ws_ref[_AB2:_AB2 + 1, 0:H])
        ab3 = ws_ref[_MISC:_MISC + 1, 0:1]
        scores = (jnp.sum(a2 * ws_ref[_AW3:_AW3 + 1, 0:H], axis=1,
                          keepdims=True) + ab3) * 10.0
        scores = jnp.where(maskf_ref[g] > 0.5, _NEG_INF, scores)

        m = jnp.max(scores, axis=0, keepdims=True)
        e = jnp.exp(scores - m)
        z = jnp.sum(e, axis=0, keepdims=True)
        log_pi = scores - m - jnp.log(z)
        pi = jnp.exp(log_pi)
        ent_ref[g] = -jnp.sum(pi * log_pi, axis=0, keepdims=True)

        aidx = idx_ref[g][N_J:N_J + 1, :]
        iota_j = jax.lax.broadcasted_iota(jnp.int32, (N_J, 1), 0)
        oh_a = (iota_j == aidx).astype(f32)
        loga_ref[g] = jnp.sum(log_pi * oh_a, axis=0, keepdims=True)

        oa = idx_ref[g][N_J + 1:N_J + 2, :]
        iota_row = jax.lax.broadcasted_iota(jnp.int32, (1, N), 1)
        oh_o = (iota_row == oa).astype(f32)
        afeat_ref[g] = jnp.dot(oh_o, h2g, preferred_element_type=f32)

        c1 = jnp.tanh(jnp.dot(h_pooled, ws_ref[_CW1:_CW1 + H, 0:H],
                              preferred_element_type=f32)
                      + ws_ref[_CB1:_CB1 + 1, 0:H])
        cb2 = ws_ref[_MISC:_MISC + 1, 1:2]
        v_ref[g] = (jnp.sum(c1 * ws_ref[_CW2:_CW2 + 1, 0:H], axis=1,
                            keepdims=True) + cb2)
        hpool_ref[g] = h_pooled


def _b3(shape):
    return pl.BlockSpec((G,) + shape, lambda b: (b, 0, 0))


def _w(shape):
    return pl.BlockSpec(shape, lambda b: (0,) * len(shape))


def kernel(x, graph_pool, padded_nei, adj, candidate, mask, mask_mch, dur,
           a_index, old_action, mch_pool,
           gW01, gb01, gW02, gb02, gW11, gb11, gW12, gb12,
           aW1, ab1, aW2, ab2, aW3, ab3, cW1, cb1, cW2, cb2):
    f32 = jnp.float32
    bf16 = jnp.bfloat16
    i32 = jnp.int32

    anode, mma = _sc_action_rows(dur, mask_mch, old_action)

    zw = jnp.zeros((D, H), f32)

    def bd(w):
        return jnp.concatenate(
            [jnp.concatenate([w, zw], 1), jnp.concatenate([zw, w], 1)], 0)

    wd = jnp.concatenate(
        [bd(gW01), bd(gW02), bd(gW11), bd(gW12)], 0).astype(bf16)

    def pad128(w):
        return jnp.concatenate([w, jnp.zeros_like(w)], 1)

    def row128(vec64):
        return jnp.concatenate([vec64, jnp.zeros((H,), f32)])[None, :]

    misc = jnp.concatenate([ab3, cb2, jnp.zeros((126,), f32)])[None, :]
    ws = jnp.concatenate(
        [pad128(aW1), pad128(aW2), pad128(cW1),
         jnp.concatenate([gb01, gb01])[None, :],
         jnp.concatenate([gb02, gb02])[None, :],
         jnp.concatenate([gb11, gb11])[None, :],
         jnp.concatenate([gb12, gb12])[None, :],
         row128(ab1), row128(ab2), row128(cb1),
         row128(aW3[:, 0]), row128(cW2[:, 0]), misc], 0)

    gm = jnp.concatenate([graph_pool, mch_pool], axis=1)[:, None, :]
    idx_all = jnp.concatenate(
        [candidate.astype(i32), a_index.astype(i32)[:, None],
         old_action.astype(i32)[:, None]], axis=1)[:, :, None]
    maskf3 = mask.astype(f32)[:, :, None]

    out_shapes = (
        jax.ShapeDtypeStruct((B, 1, 1), f32),
        jax.ShapeDtypeStruct((B, 1, 1), f32),
        jax.ShapeDtypeStruct((B, 1, 1), f32),
        jax.ShapeDtypeStruct((B, 1, H), f32),
        jax.ShapeDtypeStruct((B, 1, H), f32),
    )
    in_specs = [
        _b3((N, N)),
        _b3((N, D)),
        _b3((1, N + H)),
        _b3((N_J + 2, 1)),
        _b3((N_J, 1)),
        _w((512, 128)),
        _w((330, 128)),
    ]
    out_specs = (
        _b3((1, 1)), _b3((1, 1)), _b3((1, 1)), _b3((1, H)), _b3((1, H)),
    )
    ent, v, loga, afeat, hpool = pl.pallas_call(
        _body,
        grid=(B // G,),
        in_specs=in_specs,
        out_specs=out_specs,
        out_shape=out_shapes,
    )(adj, x, gm, idx_all, maskf3, wd, ws)

    return (ent.reshape(B), v.reshape(B, 1), loga.reshape(B),
            anode, afeat.reshape(B, H),
            mma, hpool.reshape(B, H))


# speedup vs baseline: 1.1970x; 1.0603x over previous
import functools

import jax
import jax.numpy as jnp
from jax import lax
from jax.experimental import pallas as pl
from jax.experimental.pallas import tpu as pltpu
from jax.experimental.pallas import tpu_sc as plsc

B = 32
N_J = 50
N_M = 20
N = N_J * N_M
D = 64
H = 64
G = 2

_NEG_INF = float("-inf")

_WD01, _WD02, _WD11, _WD12 = 0, 128, 256, 384
_A1C, _A1H, _A1M, _AW2, _CW1 = 0, 64, 128, 192, 256
_B01, _B02, _B11, _B12 = 320, 321, 322, 323
_AB1, _AB2, _CB1, _AW3, _CW2, _MISC = 324, 325, 326, 327, 328, 329

_RPW = 8
_NW = B // _RPW
_PAD_W = 32


def _sc_gather(table, idx):
    mesh = plsc.VectorSubcoreMesh(core_axis_name="c", subcore_axis_name="s")

    @functools.partial(
        pl.kernel, mesh=mesh,
        compiler_params=pltpu.CompilerParams(use_tc_tiling_on_sc=False),
        out_type=jax.ShapeDtypeStruct((B, _PAD_W), jnp.float32),
        scratch_types=[
            pltpu.VMEM((_RPW,), jnp.int32),
            pltpu.VMEM((_RPW, _PAD_W), jnp.float32),
            pltpu.SemaphoreType.DMA,
        ],
    )
    def k(table_hbm, idx_hbm, out_hbm, idx_v, rows_v, sem):
        wid = lax.axis_index("s") * 2 + lax.axis_index("c")

        @pl.when(wid < _NW)
        def _():
            base = wid * _RPW
            pltpu.sync_copy(idx_hbm.at[pl.ds(base, _RPW)], idx_v)
            pltpu.async_copy(table_hbm.at[idx_v], rows_v, sem).wait()
            pltpu.sync_copy(rows_v, out_hbm.at[pl.ds(base, _RPW)])

    return k(table, idx)


def _sc_action_rows(dur, mask_mch, old_action):
    f32 = jnp.float32
    i32 = jnp.int32
    dur2 = dur.reshape(B * N, N_M)
    mm_w = jax.lax.bitcast_convert_type(
        mask_mch.view(jnp.int8).reshape(B * N, N_M // 4, 4), i32)
    table = jnp.concatenate(
        [dur2, jax.lax.bitcast_convert_type(mm_w, f32),
         jnp.zeros((B * N, _PAD_W - N_M - N_M // 4), f32)], axis=1)
    ridx = old_action.astype(i32) + jnp.arange(B, dtype=i32) * N
    rows = _sc_gather(table, ridx)
    action_node = rows[:, :N_M]
    mm_back = jax.lax.bitcast_convert_type(
        jax.lax.bitcast_convert_type(rows[:, N_M:N_M + N_M // 4], i32),
        jnp.int8).reshape(B, 1, N_M)
    return action_node, mm_back != 0


def _body(adj_ref, x_ref, gm_ref, idx_ref, maskf_ref,
          wd_ref, ws_ref,
          ent_ref, v_ref, loga_ref, afeat_ref, hpool_ref):
    f32 = jnp.float32
    bf16 = jnp.bfloat16

    adjs = [adj_ref[g].astype(bf16) for g in range(G)]
    p0s = [jnp.dot(adjs[g], x_ref[g].astype(bf16),
                   preferred_element_type=f32) for g in range(G)]
    p0 = jnp.concatenate(p0s, axis=1)

    t0 = jnp.maximum(jnp.dot(p0.astype(bf16), wd_ref[_WD01:_WD01 + 128, :],
                             preferred_element_type=f32)
                     + ws_ref[_B01:_B01 + 1, :], 0.0)
    h1 = jnp.maximum(jnp.dot(t0.astype(bf16), wd_ref[_WD02:_WD02 + 128, :],
                             preferred_element_type=f32)
                     + ws_ref[_B02:_B02 + 1, :], 0.0)

    p1s = [jnp.dot(adjs[g], h1[:, g * H:(g + 1) * H].astype(bf16),
                   preferred_element_type=f32) for g in range(G)]
    p1 = jnp.concatenate(p1s, axis=1)
    t1 = jnp.maximum(jnp.dot(p1.astype(bf16), wd_ref[_WD11:_WD11 + 128, :],
                             preferred_element_type=f32)
                     + ws_ref[_B11:_B11 + 1, :], 0.0)
    h2 = jnp.maximum(jnp.dot(t1.astype(bf16), wd_ref[_WD12:_WD12 + 128, :],
                             preferred_element_type=f32)
                     + ws_ref[_B12:_B12 + 1, :], 0.0)

    for g in range(G):
        h2g = h2[:, g * H:(g + 1) * H]
        gp = gm_ref[g][:, 0:N]
        mch = gm_ref[g][:, N:N + H]
        h_pooled = jnp.dot(gp, h2g, preferred_element_type=f32)

        cand = idx_ref[g][0:N_J, :]
        iota_n = jax.lax.broadcasted_iota(jnp.int32, (N_J, N), 1)
        onehot = (iota_n == cand).astype(f32)
        cand_feat = jnp.dot(onehot, h2g, preferred_element_type=f32)

        a1 = jnp.tanh(
            jnp.dot(cand_feat, ws_ref[_A1C:_A1C + H, 0:H],
                    preferred_element_type=f32)
            + jnp.dot(h_pooled, ws_ref[_A1H:_A1H + H, 0:H],
                      preferred_element_type=f32)
            + jnp.dot(mch, ws_ref[_A1M:_A1M + H, 0:H],
                      preferred_element_type=f32)
            + ws_ref[_AB1:_AB1 + 1, 0:H])
        a2 = jnp.tanh(jnp.dot(a1, ws_ref[_AW2:_AW2 + H, 0:H],
                              preferred_element_type=f32)
                      + ws_ref[_AB2:_AB2 + 1, 0:H])
        ab3 = ws_ref[_MISC:_MISC + 1, 0:1]
        scores = (jnp.sum(a2 * ws_ref[_AW3:_AW3 + 1, 0:H], axis=1,
                          keepdims=True) + ab3) * 10.0
        scores = jnp.where(maskf_ref[g] > 0.5, _NEG_INF, scores)

        m = jnp.max(scores, axis=0, keepdims=True)
        e = jnp.exp(scores - m)
        z = jnp.sum(e, axis=0, keepdims=True)
        log_pi = scores - m - jnp.log(z)
        pi = jnp.exp(log_pi)
        ent_ref[g] = -jnp.sum(pi * log_pi, axis=0, keepdims=True)

        aidx = idx_ref[g][N_J:N_J + 1, :]
        iota_j = jax.lax.broadcasted_iota(jnp.int32, (N_J, 1), 0)
        oh_a = (iota_j == aidx).astype(f32)
        loga_ref[g] = jnp.sum(log_pi * oh_a, axis=0, keepdims=True)

        oa = idx_ref[g][N_J + 1:N_J + 2, :]
        iota_row = jax.lax.broadcasted_iota(jnp.int32, (1, N), 1)
        oh_o = (iota_row == oa).astype(f32)
        afeat_ref[g] = jnp.dot(oh_o, h2g, preferred_element_type=f32)

        c1 = jnp.tanh(jnp.dot(h_pooled, ws_ref[_CW1:_CW1 + H, 0:H],
                              preferred_element_type=f32)
                      + ws_ref[_CB1:_CB1 + 1, 0:H])
        cb2 = ws_ref[_MISC:_MISC + 1, 1:2]
        v_ref[g] = (jnp.sum(c1 * ws_ref[_CW2:_CW2 + 1, 0:H], axis=1,
                            keepdims=True) + cb2)
        hpool_ref[g] = h_pooled


def _b3(shape):
    return pl.BlockSpec((G,) + shape, lambda b: (b, 0, 0))


def _w(shape):
    return pl.BlockSpec(shape, lambda b: (0,) * len(shape))


def kernel(x, graph_pool, padded_nei, adj, candidate, mask, mask_mch, dur,
           a_index, old_action, mch_pool,
           gW01, gb01, gW02, gb02, gW11, gb11, gW12, gb12,
           aW1, ab1, aW2, ab2, aW3, ab3, cW1, cb1, cW2, cb2):
    f32 = jnp.float32
    bf16 = jnp.bfloat16
    i32 = jnp.int32

    anode, mma = _sc_action_rows(dur, mask_mch, old_action)

    zw = jnp.zeros((D, H), f32)

    def bd(w):
        return jnp.concatenate(
            [jnp.concatenate([w, zw], 1), jnp.concatenate([zw, w], 1)], 0)

    wd = jnp.concatenate(
        [bd(gW01), bd(gW02), bd(gW11), bd(gW12)], 0).astype(bf16)

    def pad128(w):
        return jnp.concatenate([w, jnp.zeros_like(w)], 1)

    def row128(vec64):
        return jnp.concatenate([vec64, jnp.zeros((H,), f32)])[None, :]

    misc = jnp.concatenate([ab3, cb2, jnp.zeros((126,), f32)])[None, :]
    ws = jnp.concatenate(
        [pad128(aW1), pad128(aW2), pad128(cW1),
         jnp.concatenate([gb01, gb01])[None, :],
         jnp.concatenate([gb02, gb02])[None, :],
         jnp.concatenate([gb11, gb11])[None, :],
         jnp.concatenate([gb12, gb12])[None, :],
         row128(ab1), row128(ab2), row128(cb1),
         row128(aW3[:, 0]), row128(cW2[:, 0]), misc], 0)

    gm = jnp.concatenate([graph_pool, mch_pool], axis=1)[:, None, :]
    idx_all = jnp.concatenate(
        [candidate.astype(i32), a_index.astype(i32)[:, None],
         old_action.astype(i32)[:, None]], axis=1)[:, :, None]
    maskf3 = mask.astype(f32)[:, :, None]

    out_shapes = (
        jax.ShapeDtypeStruct((B, 1, 1), f32),
        jax.ShapeDtypeStruct((B, 1, 1), f32),
        jax.ShapeDtypeStruct((B, 1, 1), f32),
        jax.ShapeDtypeStruct((B, 1, H), f32),
        jax.ShapeDtypeStruct((B, 1, H), f32),
    )
    in_specs = [
        _b3((N, N)),
        _b3((N, D)),
        _b3((1, N + H)),
        _b3((N_J + 2, 1)),
        _b3((N_J, 1)),
        _w((512, 128)),
        _w((330, 128)),
    ]
    out_specs = (
        _b3((1, 1)), _b3((1, 1)), _b3((1, 1)), _b3((1, H)), _b3((1, H)),
    )
    ent, v, loga, afeat, hpool = pl.pallas_call(
        _body,
        grid=(B // G,),
        in_specs=in_specs,
        out_specs=out_specs,
        out_shape=out_shapes,
    )(adj, x, gm, idx_all, maskf3, wd, ws)

    return (ent.reshape(B), v.reshape(B, 1), loga.reshape(B),
            anode, afeat.reshape(B, H),
            mma, hpool.reshape(B, H))


# speedup vs baseline: 1.1988x; 1.0015x over previous
import functools

import jax
import jax.numpy as jnp
from jax import lax
from jax.experimental import pallas as pl
from jax.experimental.pallas import tpu as pltpu
from jax.experimental.pallas import tpu_sc as plsc

B = 32
N_J = 50
N_M = 20
N = N_J * N_M
D = 64
H = 64
G = 2

_NEG_INF = float("-inf")

_WD01, _WD02, _WD11, _WD12 = 0, 128, 256, 384
_A1C, _A1H, _A1M, _AW2, _CW1 = 0, 64, 128, 192, 256
_B01, _B02, _B11, _B12 = 320, 321, 322, 323
_AB1, _AB2, _CB1, _AW3, _CW2, _MISC = 324, 325, 326, 327, 328, 329

_RPW = 8
_NW = B // _RPW
_PAD_W = 32


def _sc_gather(table, idx):
    mesh = plsc.VectorSubcoreMesh(core_axis_name="c", subcore_axis_name="s")

    @functools.partial(
        pl.kernel, mesh=mesh,
        compiler_params=pltpu.CompilerParams(use_tc_tiling_on_sc=False),
        out_type=jax.ShapeDtypeStruct((B, _PAD_W), jnp.float32),
        scratch_types=[
            pltpu.VMEM((_RPW,), jnp.int32),
            pltpu.VMEM((_RPW, _PAD_W), jnp.float32),
            pltpu.SemaphoreType.DMA,
        ],
    )
    def k(table_hbm, idx_hbm, out_hbm, idx_v, rows_v, sem):
        wid = lax.axis_index("s") * 2 + lax.axis_index("c")

        @pl.when(wid < _NW)
        def _():
            base = wid * _RPW
            pltpu.sync_copy(idx_hbm.at[pl.ds(base, _RPW)], idx_v)
            pltpu.async_copy(table_hbm.at[idx_v], rows_v, sem).wait()
            pltpu.sync_copy(rows_v, out_hbm.at[pl.ds(base, _RPW)])

    return k(table, idx)


def _sc_action_rows(dur, mask_mch, old_action):
    f32 = jnp.float32
    i32 = jnp.int32
    dur2 = dur.reshape(B * N, N_M)
    mm_w = jax.lax.bitcast_convert_type(
        mask_mch.view(jnp.int8).reshape(B * N, N_M // 4, 4), i32)
    table = jnp.concatenate(
        [dur2, jax.lax.bitcast_convert_type(mm_w, f32),
         jnp.zeros((B * N, _PAD_W - N_M - N_M // 4), f32)], axis=1)
    ridx = old_action.astype(i32) + jnp.arange(B, dtype=i32) * N
    rows = _sc_gather(table, ridx)
    action_node = rows[:, :N_M]
    mm_back = jax.lax.bitcast_convert_type(
        jax.lax.bitcast_convert_type(rows[:, N_M:N_M + N_M // 4], i32),
        jnp.int8).reshape(B, 1, N_M)
    return action_node, mm_back != 0


def _body(adj_ref, x_ref, gm_ref, idx_ref, maskf_ref,
          wd_ref, ws_ref,
          ent_ref, v_ref, loga_ref, afeat_ref, hpool_ref):
    f32 = jnp.float32
    bf16 = jnp.bfloat16

    adjs = [adj_ref[g].astype(bf16) for g in range(G)]
    p0s = [jnp.dot(adjs[g], x_ref[g].astype(bf16),
                   preferred_element_type=f32) for g in range(G)]
    p0 = jnp.concatenate(p0s, axis=1)

    t0 = jnp.maximum(jnp.dot(p0.astype(bf16), wd_ref[_WD01:_WD01 + 128, :],
                             preferred_element_type=f32)
                     + ws_ref[_B01:_B01 + 1, :], 0.0)
    h1 = jnp.maximum(jnp.dot(t0.astype(bf16), wd_ref[_WD02:_WD02 + 128, :],
                             preferred_element_type=f32)
                     + ws_ref[_B02:_B02 + 1, :], 0.0)

    p1s = [jnp.dot(adjs[g], h1[:, g * H:(g + 1) * H].astype(bf16),
                   preferred_element_type=f32) for g in range(G)]
    p1 = jnp.concatenate(p1s, axis=1)
    t1 = jnp.maximum(jnp.dot(p1.astype(bf16), wd_ref[_WD11:_WD11 + 128, :],
                             preferred_element_type=f32)
                     + ws_ref[_B11:_B11 + 1, :], 0.0)
    h2 = jnp.maximum(jnp.dot(t1.astype(bf16), wd_ref[_WD12:_WD12 + 128, :],
                             preferred_element_type=f32)
                     + ws_ref[_B12:_B12 + 1, :], 0.0)

    for g in range(G):
        h2g = h2[:, g * H:(g + 1) * H]
        gp = gm_ref[g][:, 0:N]
        mch = gm_ref[g][:, N:N + H]
        h_pooled = jnp.dot(gp, h2g, preferred_element_type=f32)

        cand = idx_ref[g][0:N_J, :]
        iota_n = jax.lax.broadcasted_iota(jnp.int32, (N_J, N), 1)
        onehot = (iota_n == cand).astype(f32)
        cand_feat = jnp.dot(onehot, h2g, preferred_element_type=f32)

        a1 = jnp.tanh(
            jnp.dot(cand_feat, ws_ref[_A1C:_A1C + H, 0:H],
                    preferred_element_type=f32)
            + jnp.dot(h_pooled, ws_ref[_A1H:_A1H + H, 0:H],
                      preferred_element_type=f32)
            + jnp.dot(mch, ws_ref[_A1M:_A1M + H, 0:H],
                      preferred_element_type=f32)
            + ws_ref[_AB1:_AB1 + 1, 0:H])
        a2 = jnp.tanh(jnp.dot(a1, ws_ref[_AW2:_AW2 + H, 0:H],
                              preferred_element_type=f32)
                      + ws_ref[_AB2:_AB2 + 1, 0:H])
        ab3 = ws_ref[_MISC:_MISC + 1, 0:1]
        scores = (jnp.sum(a2 * ws_ref[_AW3:_AW3 + 1, 0:H], axis=1,
                          keepdims=True) + ab3) * 10.0
        scores = jnp.where(maskf_ref[g] > 0.5, _NEG_INF, scores)

        m = jnp.max(scores, axis=0, keepdims=True)
        e = jnp.exp(scores - m)
        z = jnp.sum(e, axis=0, keepdims=True)
        log_pi = scores - m - jnp.log(z)
        pi = jnp.exp(log_pi)
        ent_ref[g] = -jnp.sum(pi * log_pi, axis=0, keepdims=True)

        aidx = idx_ref[g][N_J:N_J + 1, :]
        iota_j = jax.lax.broadcasted_iota(jnp.int32, (N_J, 1), 0)
        oh_a = (iota_j == aidx).astype(f32)
        loga_ref[g] = jnp.sum(log_pi * oh_a, axis=0, keepdims=True)

        oa = idx_ref[g][N_J + 1:N_J + 2, :]
        iota_row = jax.lax.broadcasted_iota(jnp.int32, (1, N), 1)
        oh_o = (iota_row == oa).astype(f32)
        afeat_ref[g] = jnp.dot(oh_o, h2g, preferred_element_type=f32)

        c1 = jnp.tanh(jnp.dot(h_pooled, ws_ref[_CW1:_CW1 + H, 0:H],
                              preferred_element_type=f32)
                      + ws_ref[_CB1:_CB1 + 1, 0:H])
        cb2 = ws_ref[_MISC:_MISC + 1, 1:2]
        v_ref[g] = (jnp.sum(c1 * ws_ref[_CW2:_CW2 + 1, 0:H], axis=1,
                            keepdims=True) + cb2)
        hpool_ref[g] = h_pooled


def _b3(shape):
    return pl.BlockSpec((G,) + shape, lambda b: (b, 0, 0))


def _w(shape):
    return pl.BlockSpec(shape, lambda b: (0,) * len(shape))


def kernel(x, graph_pool, padded_nei, adj, candidate, mask, mask_mch, dur,
           a_index, old_action, mch_pool,
           gW01, gb01, gW02, gb02, gW11, gb11, gW12, gb12,
           aW1, ab1, aW2, ab2, aW3, ab3, cW1, cb1, cW2, cb2):
    f32 = jnp.float32
    bf16 = jnp.bfloat16
    i32 = jnp.int32

    anode, mma = _sc_action_rows(dur, mask_mch, old_action)

    zw = jnp.zeros((D, H), f32)

    def bd(w):
        return jnp.concatenate(
            [jnp.concatenate([w, zw], 1), jnp.concatenate([zw, w], 1)], 0)

    wd = jnp.concatenate(
        [bd(gW01), bd(gW02), bd(gW11), bd(gW12)], 0).astype(bf16)

    def pad128(w):
        return jnp.concatenate([w, jnp.zeros_like(w)], 1)

    def row128(vec64):
        return jnp.concatenate([vec64, jnp.zeros((H,), f32)])[None, :]

    misc = jnp.concatenate([ab3, cb2, jnp.zeros((126,), f32)])[None, :]
    ws = jnp.concatenate(
        [pad128(aW1), pad128(aW2), pad128(cW1),
         jnp.concatenate([gb01, gb01])[None, :],
         jnp.concatenate([gb02, gb02])[None, :],
         jnp.concatenate([gb11, gb11])[None, :],
         jnp.concatenate([gb12, gb12])[None, :],
         row128(ab1), row128(ab2), row128(cb1),
         row128(aW3[:, 0]), row128(cW2[:, 0]), misc], 0)

    gm = jnp.concatenate([graph_pool, mch_pool], axis=1)[:, None, :]
    idx_all = jnp.concatenate(
        [candidate.astype(i32), a_index.astype(i32)[:, None],
         old_action.astype(i32)[:, None]], axis=1)[:, :, None]
    maskf3 = mask.astype(f32)[:, :, None]

    out_shapes = (
        jax.ShapeDtypeStruct((B, 1, 1), f32),
        jax.ShapeDtypeStruct((B, 1, 1), f32),
        jax.ShapeDtypeStruct((B, 1, 1), f32),
        jax.ShapeDtypeStruct((B, 1, H), f32),
        jax.ShapeDtypeStruct((B, 1, H), f32),
    )
    in_specs = [
        _b3((N, N)),
        _b3((N, D)),
        _b3((1, N + H)),
        _b3((N_J + 2, 1)),
        _b3((N_J, 1)),
        _w((512, 128)),
        _w((330, 128)),
    ]
    out_specs = (
        _b3((1, 1)), _b3((1, 1)), _b3((1, 1)), _b3((1, H)), _b3((1, H)),
    )
    ent, v, loga, afeat, hpool = pl.pallas_call(
        _body,
        grid=(B // G,),
        in_specs=in_specs,
        out_specs=out_specs,
        out_shape=out_shapes,
        compiler_params=pltpu.CompilerParams(
            dimension_semantics=("arbitrary",)),
    )(adj, x, gm, idx_all, maskf3, wd, ws)

    return (ent.reshape(B), v.reshape(B, 1), loga.reshape(B),
            anode, afeat.reshape(B, H),
            mma, hpool.reshape(B, H))
